# Initial kernel scaffold; baseline (speedup 1.0000x reference)
import jax
import jax.numpy as jnp
from jax import lax
from jax.experimental import pallas as pl
from jax.experimental.pallas import tpu as pltpu

B, N, C = 8, 20000, 80
CONF_THR = 0.25
IOU_THR = 0.5
MAX_OUT = 100
NEG = jnp.float32(-1e9)

BN = 2000


def _score_decode_kernel(reg_ref, log_ref, anc_ref, out_ref):
    probs = jax.nn.sigmoid(log_ref[0])
    mx = jnp.max(probs, axis=-1)
    cls = jnp.argmax(probs, axis=-1)

    a = anc_ref[0]
    r = reg_ref[0]
    ax, ay, aw, ah = a[0], a[1], a[2], a[3]
    dx, dy, dw, dh = r[0], r[1], r[2], r[3]
    cx = ax + dx * aw
    cy = ay + dy * ah
    w = aw * jnp.exp(dw)
    h = ah * jnp.exp(dh)
    x1 = cx - w * 0.5
    y1 = cy - h * 0.5
    x2 = cx + w * 0.5
    y2 = cy + h * 0.5
    area = jnp.maximum(x2 - x1, 0.0) * jnp.maximum(y2 - y1, 0.0)
    s0 = jnp.where(mx > CONF_THR, mx, NEG)

    out_ref[0, 0, :] = s0
    out_ref[0, 1, :] = mx
    out_ref[0, 2, :] = cls.astype(jnp.float32)
    out_ref[0, 3, :] = x1
    out_ref[0, 4, :] = y1
    out_ref[0, 5, :] = x2
    out_ref[0, 6, :] = y2
    out_ref[0, 7, :] = area


def _nms_kernel(s0_ref, sc_ref, cl_ref, x1_ref, y1_ref, x2_ref, y2_ref,
                ar_ref, nd_ref, det_ref, s_scr):
    s_scr[...] = s0_ref[...]
    iota = lax.broadcasted_iota(jnp.int32, (B, N), 1)
    big = jnp.int32(N + 1)

    def body(i, cnt):
        s = s_scr[...]
        m = jnp.max(s, axis=1, keepdims=True)
        idxv = jnp.where(s == m, iota, big)
        idx = jnp.min(idxv, axis=1, keepdims=True)
        ok = m > NEG * 0.5
        sel = iota == idx

        x1 = x1_ref[...]
        y1 = y1_ref[...]
        x2 = x2_ref[...]
        y2 = y2_ref[...]
        bx1 = jnp.sum(jnp.where(sel, x1, 0.0), axis=1, keepdims=True)
        by1 = jnp.sum(jnp.where(sel, y1, 0.0), axis=1, keepdims=True)
        bx2 = jnp.sum(jnp.where(sel, x2, 0.0), axis=1, keepdims=True)
        by2 = jnp.sum(jnp.where(sel, y2, 0.0), axis=1, keepdims=True)
        bcl = jnp.sum(jnp.where(sel, cl_ref[...], 0.0), axis=1, keepdims=True)
        bsc = jnp.sum(jnp.where(sel, sc_ref[...], 0.0), axis=1, keepdims=True)

        xx1 = jnp.maximum(bx1, x1)
        yy1 = jnp.maximum(by1, y1)
        xx2 = jnp.minimum(bx2, x2)
        yy2 = jnp.minimum(by2, y2)
        inter = jnp.maximum(xx2 - xx1, 0.0) * jnp.maximum(yy2 - yy1, 0.0)
        area_b = jnp.maximum(bx2 - bx1, 0.0) * jnp.maximum(by2 - by1, 0.0)
        iou = inter / (area_b + ar_ref[...] - inter + 1e-9)
        suppress = (iou >= IOU_THR) & ok
        s_scr[...] = jnp.where(suppress | sel, NEG, s)

        okf = ok.astype(jnp.float32)
        row = jnp.concatenate(
            [jnp.where(ok, bcl, -1.0), bsc * okf, bx1 * okf, by1 * okf,
             bx2 * okf, by2 * okf], axis=1)
        det_ref[pl.ds(i, 1)] = row[None]
        return cnt + ok.astype(jnp.int32)

    cnt = lax.fori_loop(0, MAX_OUT, body, jnp.zeros((B, 1), jnp.int32))
    nd_ref[...] = cnt


def kernel(regress, logits, anchors):
    reg_t = jnp.transpose(regress, (0, 2, 1))
    anc_t = jnp.transpose(anchors, (0, 2, 1))

    planes = pl.pallas_call(
        _score_decode_kernel,
        grid=(B, N // BN),
        in_specs=[
            pl.BlockSpec((1, 4, BN), lambda b, n: (b, 0, n)),
            pl.BlockSpec((1, BN, C), lambda b, n: (b, n, 0)),
            pl.BlockSpec((1, 4, BN), lambda b, n: (b, 0, n)),
        ],
        out_specs=pl.BlockSpec((1, 8, BN), lambda b, n: (b, 0, n)),
        out_shape=jax.ShapeDtypeStruct((B, 8, N), jnp.float32),
    )(reg_t, logits, anc_t)

    s0 = planes[:, 0, :]
    sc = planes[:, 1, :]
    cl = planes[:, 2, :]
    x1 = planes[:, 3, :]
    y1 = planes[:, 4, :]
    x2 = planes[:, 5, :]
    y2 = planes[:, 6, :]
    ar = planes[:, 7, :]

    num_dets, dets_t = pl.pallas_call(
        _nms_kernel,
        out_shape=(
            jax.ShapeDtypeStruct((B, 1), jnp.int32),
            jax.ShapeDtypeStruct((MAX_OUT, B, 6), jnp.float32),
        ),
        scratch_shapes=[pltpu.VMEM((B, N), jnp.float32)],
    )(s0, sc, cl, x1, y1, x2, y2, ar)

    dets = jnp.transpose(dets_t, (1, 0, 2))
    return num_dets[:, 0], dets


# speedup vs baseline: 10.5182x; 10.5182x over previous
import jax
import jax.numpy as jnp
from jax import lax
from jax.experimental import pallas as pl
from jax.experimental.pallas import tpu as pltpu

B, N, C = 8, 20000, 80
CONF_THR = 0.25
IOU_THR = 0.5
MAX_OUT = 100
NEG = -1e9

BN = 2000


def _score_decode_kernel(reg_ref, log_ref, anc_ref, out_ref):
    probs = jax.nn.sigmoid(log_ref[0])
    mx = jnp.max(probs, axis=-1)
    cls = jnp.argmax(probs, axis=-1)

    a = anc_ref[0]
    r = reg_ref[0]
    ax, ay, aw, ah = a[0], a[1], a[2], a[3]
    dx, dy, dw, dh = r[0], r[1], r[2], r[3]
    cx = ax + dx * aw
    cy = ay + dy * ah
    w = aw * jnp.exp(dw)
    h = ah * jnp.exp(dh)
    x1 = cx - w * 0.5
    y1 = cy - h * 0.5
    x2 = cx + w * 0.5
    y2 = cy + h * 0.5
    area = jnp.maximum(x2 - x1, 0.0) * jnp.maximum(y2 - y1, 0.0)
    s0 = jnp.where(mx > CONF_THR, mx, NEG)

    out_ref[0, 0, :] = s0
    out_ref[0, 1, :] = mx
    out_ref[0, 2, :] = cls.astype(jnp.float32)
    out_ref[0, 3, :] = x1
    out_ref[0, 4, :] = y1
    out_ref[0, 5, :] = x2
    out_ref[0, 6, :] = y2
    out_ref[0, 7, :] = area


def _nms_kernel(s0_ref, sc_ref, cl_ref, x1_ref, y1_ref, x2_ref, y2_ref,
                ar_ref, nd_ref, det_ref, s_scr):
    s_scr[...] = s0_ref[...]
    iota = lax.broadcasted_iota(jnp.int32, (B, N), 1)
    big = jnp.int32(N + 1)

    def body(i, cnt):
        s = s_scr[...]
        m = jnp.max(s, axis=1, keepdims=True)
        idxv = jnp.where(s == m, iota, big)
        idx = jnp.min(idxv, axis=1, keepdims=True)
        ok = m > NEG * 0.5
        sel = iota == idx

        x1 = x1_ref[...]
        y1 = y1_ref[...]
        x2 = x2_ref[...]
        y2 = y2_ref[...]
        bx1 = jnp.sum(jnp.where(sel, x1, 0.0), axis=1, keepdims=True)
        by1 = jnp.sum(jnp.where(sel, y1, 0.0), axis=1, keepdims=True)
        bx2 = jnp.sum(jnp.where(sel, x2, 0.0), axis=1, keepdims=True)
        by2 = jnp.sum(jnp.where(sel, y2, 0.0), axis=1, keepdims=True)
        bcl = jnp.sum(jnp.where(sel, cl_ref[...], 0.0), axis=1, keepdims=True)
        bsc = jnp.sum(jnp.where(sel, sc_ref[...], 0.0), axis=1, keepdims=True)

        xx1 = jnp.maximum(bx1, x1)
        yy1 = jnp.maximum(by1, y1)
        xx2 = jnp.minimum(bx2, x2)
        yy2 = jnp.minimum(by2, y2)
        inter = jnp.maximum(xx2 - xx1, 0.0) * jnp.maximum(yy2 - yy1, 0.0)
        area_b = jnp.maximum(bx2 - bx1, 0.0) * jnp.maximum(by2 - by1, 0.0)
        iou = inter / (area_b + ar_ref[...] - inter + 1e-9)
        suppress = (iou >= IOU_THR) & ok
        s_scr[...] = jnp.where(suppress | sel, NEG, s)

        okf = ok.astype(jnp.float32)
        row = jnp.concatenate(
            [jnp.where(ok, bcl, -1.0), bsc * okf, bx1 * okf, by1 * okf,
             bx2 * okf, by2 * okf], axis=1)
        det_ref[pl.ds(i, 1)] = row[None]
        return cnt + ok.astype(jnp.int32)

    cnt = lax.fori_loop(0, MAX_OUT, body, jnp.zeros((B, 1), jnp.int32))
    nd_ref[...] = cnt


def kernel(regress, logits, anchors):
    reg_t = jnp.transpose(regress, (0, 2, 1))
    anc_t = jnp.transpose(anchors, (0, 2, 1))

    planes = pl.pallas_call(
        _score_decode_kernel,
        grid=(B,),
        in_specs=[
            pl.BlockSpec((1, 4, N), lambda b: (b, 0, 0)),
            pl.BlockSpec((1, N, C), lambda b: (b, 0, 0)),
            pl.BlockSpec((1, 4, N), lambda b: (b, 0, 0)),
        ],
        out_specs=pl.BlockSpec((1, 8, N), lambda b: (b, 0, 0)),
        out_shape=jax.ShapeDtypeStruct((B, 8, N), jnp.float32),
    )(reg_t, logits, anc_t)

    s0 = planes[:, 0, :]
    sc = planes[:, 1, :]
    cl = planes[:, 2, :]
    x1 = planes[:, 3, :]
    y1 = planes[:, 4, :]
    x2 = planes[:, 5, :]
    y2 = planes[:, 6, :]
    ar = planes[:, 7, :]

    num_dets, dets_t = pl.pallas_call(
        _nms_kernel,
        out_shape=(
            jax.ShapeDtypeStruct((B, 1), jnp.int32),
            jax.ShapeDtypeStruct((MAX_OUT, B, 6), jnp.float32),
        ),
        scratch_shapes=[pltpu.VMEM((B, N), jnp.float32)],
    )(s0, sc, cl, x1, y1, x2, y2, ar)

    dets = jnp.transpose(dets_t, (1, 0, 2))
    return num_dets[:, 0], dets
